# Initial kernel scaffold; baseline (speedup 1.0000x reference)
import jax
import jax.numpy as jnp
from jax.experimental import pallas as pl


def kernel(x, coefficients_vect, scaling_coeffs_vect):
    raise NotImplementedError("write your pallas kernel here")


# speedup vs baseline: 349.1017x; 349.1017x over previous
import dataclasses
import functools

import jax
import jax.numpy as jnp
from jax.experimental import pallas as pl
from jax.experimental.pallas import tpu as pltpu
from jax.experimental.pallas import tpu_sc as plsc

_NUM_ACT = 96
_SIZE = 51
_RANGE = 4.0
_GRID = 2.0 * _RANGE / (_SIZE - 1)
_INV_GRID = (_SIZE - 1) / (2.0 * _RANGE)
_SHIFT = float(_SIZE // 2)
_TMAX = float(_SIZE - 2)
_HALF_GRID = _GRID / 2.0

_BLOCKW = 16384
_LANES = 16


def _spline_sc(x2, table, scale_b, inv_b, base_b):
    nblk = x2.shape[0]
    mesh = plsc.VectorSubcoreMesh(
        core_axis_name="core", subcore_axis_name="subcore"
    )

    cp = pltpu.CompilerParams()
    if "needs_layout_passes" in pltpu.CompilerParams.__dataclass_fields__:
        cp = dataclasses.replace(cp, needs_layout_passes=False)

    @functools.partial(
        pl.kernel,
        mesh=mesh,
        out_type=jax.ShapeDtypeStruct(x2.shape, jnp.float32),
        scratch_types=[pltpu.VMEM((_NUM_ACT * _SIZE,), jnp.float32)],
        compiler_params=cp,
    )
    def run(x_hbm, tab_hbm, scale_hbm, inv_hbm, base_hbm, out_hbm, tab_vmem):
        pltpu.sync_copy(tab_hbm, tab_vmem)

        def body(x_vmem, scale_vmem, inv_vmem, base_vmem, out_vmem):
            s = scale_vmem[0, :]
            inv = inv_vmem[0, :]
            base = base_vmem[0, :]

            @pl.loop(0, _BLOCKW, step=_LANES)
            def _(c1):
                v = x_vmem[0, pl.ds(c1, _LANES)]
                tt = v * s * _INV_GRID + _SHIFT
                tc = jnp.minimum(jnp.maximum(tt, 0.0), _TMAX)
                fi = tc.astype(jnp.int32)
                frac = tt - fi.astype(jnp.float32)
                idx = base + fi
                g0 = plsc.load_gather(tab_vmem, [idx])
                g1 = plsc.load_gather(tab_vmem, [idx + 1])
                r = (g0 + frac * (g1 - g0) - _HALF_GRID) * inv
                out_vmem[0, pl.ds(c1, _LANES)] = r

        pltpu.emit_pipeline(
            body,
            grid=(nblk,),
            in_specs=[
                pl.BlockSpec((1, _BLOCKW), lambda i: (i, 0)),
                pl.BlockSpec((1, _LANES), lambda i: (i, 0)),
                pl.BlockSpec((1, _LANES), lambda i: (i, 0)),
                pl.BlockSpec((1, _LANES), lambda i: (i, 0)),
            ],
            out_specs=[pl.BlockSpec((1, _BLOCKW), lambda i: (i, 0))],
            core_axis_name=("core", "subcore"),
            dimension_semantics=(pltpu.PARALLEL,),
        )(x_hbm, scale_hbm, inv_hbm, base_hbm, out_hbm)

    return run(x2, table, scale_b, inv_b, base_b)


def kernel(x, coefficients_vect, scaling_coeffs_vect):
    b, c, h, w = x.shape
    row = h * w
    blocks_per_row = row // _BLOCKW
    nblk = b * c * blocks_per_row

    x2 = x.reshape(nblk, _BLOCKW)
    s = scaling_coeffs_vect.reshape(c)
    ch = (jnp.arange(nblk, dtype=jnp.int32) // blocks_per_row) % c
    scale_b = jnp.broadcast_to(s[ch][:, None], (nblk, _LANES))
    inv_b = jnp.broadcast_to((1.0 / s)[ch][:, None], (nblk, _LANES))
    base_b = jnp.broadcast_to(
        (ch * _SIZE)[:, None].astype(jnp.int32), (nblk, _LANES)
    )

    out2 = _spline_sc(x2, coefficients_vect, scale_b, inv_b, base_b)
    return out2.reshape(x.shape)


# speedup vs baseline: 1513.4131x; 4.3352x over previous
import dataclasses
import functools

import jax
import jax.numpy as jnp
from jax.experimental import pallas as pl
from jax.experimental.pallas import tpu as pltpu
from jax.experimental.pallas import tpu_sc as plsc

_NUM_ACT = 96
_SIZE = 51
_RANGE = 4.0
_GRID = 2.0 * _RANGE / (_SIZE - 1)
_INV_GRID = (_SIZE - 1) / (2.0 * _RANGE)
_SHIFT = float(_SIZE // 2)
_TMAX = float(_SIZE - 2)
_HALF_GRID = _GRID / 2.0

_BLOCKW = 16384
_LANES = 16
_UNROLL = 8


def _spline_sc(x2, tab0, tab1, meta):
    nblk = x2.shape[0]
    tab_len = tab0.shape[0]
    mesh = plsc.VectorSubcoreMesh(
        core_axis_name="core", subcore_axis_name="subcore"
    )

    cp = pltpu.CompilerParams()
    if "needs_layout_passes" in pltpu.CompilerParams.__dataclass_fields__:
        cp = dataclasses.replace(cp, needs_layout_passes=False)

    @functools.partial(
        pl.kernel,
        mesh=mesh,
        out_type=jax.ShapeDtypeStruct(x2.shape, jnp.float32),
        scratch_types=[
            pltpu.VMEM((tab_len,), jnp.float32),
            pltpu.VMEM((tab_len,), jnp.float32),
        ],
        compiler_params=cp,
    )
    def run(x_hbm, tab0_hbm, tab1_hbm, meta_hbm, out_hbm, tab0_v, tab1_v):
        pltpu.sync_copy(tab0_hbm, tab0_v)
        pltpu.sync_copy(tab1_hbm, tab1_v)

        def body(x_vmem, meta_vmem, out_vmem):
            sg = meta_vmem[0, 0:_LANES]
            inv = meta_vmem[0, _LANES : 2 * _LANES]
            base = meta_vmem[0, 2 * _LANES : 3 * _LANES].astype(jnp.int32)
            hginv = meta_vmem[0, 3 * _LANES : 4 * _LANES]

            @plsc.parallel_loop(0, _BLOCKW, _LANES, unroll=_UNROLL)
            def _(c1):
                v = x_vmem[0, pl.ds(c1, _LANES)]
                tt = v * sg + _SHIFT
                tc = jnp.minimum(jnp.maximum(tt, 0.0), _TMAX)
                fi = tc.astype(jnp.int32)
                frac = tt - fi.astype(jnp.float32)
                idx = base + fi
                g0 = plsc.load_gather(tab0_v, [idx])
                g1 = plsc.load_gather(tab1_v, [idx])
                r = (g0 + frac * (g1 - g0)) * inv - hginv
                out_vmem[0, pl.ds(c1, _LANES)] = r

        pltpu.emit_pipeline(
            body,
            grid=(nblk,),
            in_specs=[
                pl.BlockSpec((1, _BLOCKW), lambda i: (i, 0)),
                pl.BlockSpec((1, 4 * _LANES), lambda i: (i, 0)),
            ],
            out_specs=[pl.BlockSpec((1, _BLOCKW), lambda i: (i, 0))],
            core_axis_name=("core", "subcore"),
            dimension_semantics=(pltpu.PARALLEL,),
        )(x_hbm, meta_hbm, out_hbm)

    return run(x2, tab0, tab1, meta)


def kernel(x, coefficients_vect, scaling_coeffs_vect):
    b, c, h, w = x.shape
    row = h * w
    blocks_per_row = row // _BLOCKW
    nblk = b * c * blocks_per_row

    x2 = x.reshape(nblk, _BLOCKW)
    tab1 = jnp.concatenate([coefficients_vect[1:], coefficients_vect[-1:]])

    s = scaling_coeffs_vect.reshape(c)
    inv = 1.0 / s
    ch = (jnp.arange(nblk, dtype=jnp.int32) // blocks_per_row) % c
    meta_rows = jnp.stack(
        [
            s * jnp.float32(_INV_GRID),
            inv,
            (jnp.arange(c, dtype=jnp.int32) * _SIZE).astype(jnp.float32),
            inv * jnp.float32(_HALF_GRID),
        ],
        axis=1,
    )
    meta = jnp.broadcast_to(meta_rows[ch][:, :, None], (nblk, 4, _LANES))
    meta = meta.reshape(nblk, 4 * _LANES)

    out2 = _spline_sc(x2, coefficients_vect, tab1, meta)
    return out2.reshape(x.shape)


# speedup vs baseline: 2416.6586x; 1.5968x over previous
import dataclasses
import functools

import jax
import jax.numpy as jnp
from jax.experimental import pallas as pl
from jax.experimental.pallas import tpu as pltpu
from jax.experimental.pallas import tpu_sc as plsc

_NUM_ACT = 96
_SIZE = 51
_RANGE = 4.0
_GRID = 2.0 * _RANGE / (_SIZE - 1)
_INV_GRID = (_SIZE - 1) / (2.0 * _RANGE)
_SHIFT = float(_SIZE // 2)
_TMAX = float(_SIZE - 2)
_HALF_GRID = _GRID / 2.0

_RBLK = 64
_LANES = 16


def _spline_sc(x3, tab0, tab1, sg):
    nrow, h, w = x3.shape
    nch = sg.shape[0]
    tab_len = tab0.shape[0]
    mesh = plsc.VectorSubcoreMesh(
        core_axis_name="core", subcore_axis_name="subcore"
    )

    cp = pltpu.CompilerParams(use_tc_tiling_on_sc=True)
    if "needs_layout_passes" in pltpu.CompilerParams.__dataclass_fields__:
        cp = dataclasses.replace(cp, needs_layout_passes=False)

    @functools.partial(
        pl.kernel,
        mesh=mesh,
        out_type=jax.ShapeDtypeStruct(x3.shape, jnp.float32),
        scratch_types=[
            pltpu.VMEM((tab_len,), jnp.float32),
            pltpu.VMEM((tab_len,), jnp.float32),
            pltpu.VMEM((nch,), jnp.float32),
        ],
        compiler_params=cp,
    )
    def run(x_hbm, tab0_hbm, tab1_hbm, sg_hbm, out_hbm, tab0_v, tab1_v, sg_v):
        pltpu.sync_copy(tab0_hbm, tab0_v)
        pltpu.sync_copy(tab1_hbm, tab1_v)
        pltpu.sync_copy(sg_hbm, sg_v)

        def body(idxs, x_vmem, out_vmem):
            ch = jax.lax.rem(idxs[0], nch)
            chv = jnp.full((_LANES,), ch, jnp.int32)
            sgv = plsc.load_gather(sg_v, [chv])
            basev = chv * _SIZE

            @plsc.parallel_loop(0, _RBLK, 1, unroll=2)
            def _(r):
                for c0 in range(0, w, _LANES):
                    v = x_vmem[0, r, pl.ds(c0, _LANES)]
                    tt = v * sgv + _SHIFT
                    tc = jnp.minimum(jnp.maximum(tt, 0.0), _TMAX)
                    fi = tc.astype(jnp.int32)
                    frac = tt - fi.astype(jnp.float32)
                    idx = basev + fi
                    g0 = plsc.load_gather(tab0_v, [idx])
                    g1 = plsc.load_gather(tab1_v, [idx])
                    out_vmem[0, r, pl.ds(c0, _LANES)] = g0 + frac * (g1 - g0)

        pltpu.emit_pipeline(
            body,
            grid=(nrow, h // _RBLK),
            in_specs=[pl.BlockSpec((1, _RBLK, w), lambda i, j: (i, j, 0))],
            out_specs=[pl.BlockSpec((1, _RBLK, w), lambda i, j: (i, j, 0))],
            core_axis_name=("core", "subcore"),
            dimension_semantics=(pltpu.PARALLEL, pltpu.PARALLEL),
            _explicit_indices=True,
        )(x_hbm, out_hbm)

    return run(x3, tab0, tab1, sg)


def kernel(x, coefficients_vect, scaling_coeffs_vect):
    b, c, h, w = x.shape
    x3 = x.reshape(b * c, h, w)

    s = scaling_coeffs_vect.reshape(c)
    tabf = (
        (coefficients_vect.reshape(c, _SIZE) - jnp.float32(_HALF_GRID))
        / s[:, None]
    ).reshape(-1)
    tab1f = jnp.concatenate([tabf[1:], tabf[-1:]])
    sg = s * jnp.float32(_INV_GRID)

    out3 = _spline_sc(x3, tabf, tab1f, sg)
    return out3.reshape(x.shape)
